# baseline (speedup 1.0000x reference)
import jax
import jax.numpy as jnp
from jax.experimental import pallas as pl
from jax.experimental.pallas import tpu as pltpu

_C = 1024
_K = 8
_D = 256

_BB = 1024
_CC = 256


def _normalize_kernel(z_ref, p_ref, zn_ref, pn_ref):
    zt = z_ref[...]
    zn_ref[...] = (zt * jax.lax.rsqrt(
        jnp.maximum(jnp.sum(zt * zt, axis=1, keepdims=True), 1e-24))
    ).astype(jnp.bfloat16)
    pt = p_ref[...]
    pn_ref[...] = (pt * jax.lax.rsqrt(
        jnp.maximum(jnp.sum(pt * pt, axis=1, keepdims=True), 1e-24))
    ).astype(jnp.bfloat16)


def _fused_kernel(zn_ref, y_ref, pn_ref, pos_ref, neg_ref):
    j = pl.program_id(1)
    zn = zn_ref[...]

    m = jax.lax.dot_general(
        pn_ref[0], zn, dimension_numbers=(((1,), (1,)), ((), ())),
        preferred_element_type=jnp.float32)
    for k in range(1, _K):
        m = jnp.maximum(m, jax.lax.dot_general(
            pn_ref[k], zn, dimension_numbers=(((1,), (1,)), ((), ())),
            preferred_element_type=jnp.float32))

    cls = j * _CC + jax.lax.broadcasted_iota(jnp.int32, (_CC, _BB), 0)
    same = cls == y_ref[...][None, :]

    ninf = jnp.float32(-jnp.inf)
    pos_c = jnp.max(jnp.where(same, m, ninf), axis=0)
    neg_c = jnp.max(jnp.where(same, ninf, m), axis=0)

    @pl.when(j == 0)
    def _init():
        pos_ref[...] = pos_c
        neg_ref[...] = neg_c

    @pl.when(j != 0)
    def _acc():
        pos_ref[...] = jnp.maximum(pos_ref[...], pos_c)
        neg_ref[...] = jnp.maximum(neg_ref[...], neg_c)


def kernel(z, y, protos):
    B = z.shape[0]
    Pt = protos.transpose(1, 0, 2).reshape(_K * _C, _D)

    zn, pn = pl.pallas_call(
        _normalize_kernel,
        out_shape=[
            jax.ShapeDtypeStruct((B, _D), jnp.bfloat16),
            jax.ShapeDtypeStruct((_K * _C, _D), jnp.bfloat16),
        ],
    )(z, Pt)
    pn = pn.reshape(_K, _C, _D)

    grid = (B // _BB, _C // _CC)
    pos, neg = pl.pallas_call(
        _fused_kernel,
        grid=grid,
        in_specs=[
            pl.BlockSpec((_BB, _D), lambda i, j: (i, 0)),
            pl.BlockSpec((_BB,), lambda i, j: (i,)),
            pl.BlockSpec((_K, _CC, _D), lambda i, j: (0, j, 0)),
        ],
        out_specs=[
            pl.BlockSpec((_BB,), lambda i, j: (i,)),
            pl.BlockSpec((_BB,), lambda i, j: (i,)),
        ],
        out_shape=[
            jax.ShapeDtypeStruct((B,), jnp.float32),
            jax.ShapeDtypeStruct((B,), jnp.float32),
        ],
        compiler_params=pltpu.CompilerParams(
            dimension_semantics=("parallel", "arbitrary")),
    )(zn, y, pn)
    return (pos, neg)


# speedup vs baseline: 1.0061x; 1.0061x over previous
import jax
import jax.numpy as jnp
from jax.experimental import pallas as pl
from jax.experimental.pallas import tpu as pltpu

_C = 1024
_K = 8
_D = 256

_BB = 1024
_CC = 256


def _normalize_kernel(z_ref, p_ref, zn_ref, pn_ref):
    zt = z_ref[...]
    zn_ref[...] = (zt * jax.lax.rsqrt(
        jnp.maximum(jnp.sum(zt * zt, axis=1, keepdims=True), 1e-24))
    ).astype(jnp.bfloat16)
    for k in range(_K):
        pt = p_ref[:, k * _D:(k + 1) * _D]
        pn_ref[k] = (pt * jax.lax.rsqrt(
            jnp.maximum(jnp.sum(pt * pt, axis=1, keepdims=True), 1e-24))
        ).astype(jnp.bfloat16)


def _fused_kernel(zn_ref, y_ref, pn_ref, pos_ref, neg_ref):
    j = pl.program_id(1)
    zn = zn_ref[...]

    m = jax.lax.dot_general(
        pn_ref[0], zn, dimension_numbers=(((1,), (1,)), ((), ())),
        preferred_element_type=jnp.float32)
    for k in range(1, _K):
        m = jnp.maximum(m, jax.lax.dot_general(
            pn_ref[k], zn, dimension_numbers=(((1,), (1,)), ((), ())),
            preferred_element_type=jnp.float32))

    cls = j * _CC + jax.lax.broadcasted_iota(jnp.int32, (_CC, _BB), 0)
    same = cls == y_ref[...][None, :]

    ninf = jnp.float32(-jnp.inf)
    pos_c = jnp.max(jnp.where(same, m, ninf), axis=0)
    neg_c = jnp.max(jnp.where(same, ninf, m), axis=0)

    @pl.when(j == 0)
    def _init():
        pos_ref[...] = pos_c
        neg_ref[...] = neg_c

    @pl.when(j != 0)
    def _acc():
        pos_ref[...] = jnp.maximum(pos_ref[...], pos_c)
        neg_ref[...] = jnp.maximum(neg_ref[...], neg_c)


def kernel(z, y, protos):
    B = z.shape[0]
    zn, pn = pl.pallas_call(
        _normalize_kernel,
        out_shape=[
            jax.ShapeDtypeStruct((B, _D), jnp.bfloat16),
            jax.ShapeDtypeStruct((_K, _C, _D), jnp.bfloat16),
        ],
    )(z, protos.reshape(_C, _K * _D))

    grid = (B // _BB, _C // _CC)
    pos, neg = pl.pallas_call(
        _fused_kernel,
        grid=grid,
        in_specs=[
            pl.BlockSpec((_BB, _D), lambda i, j: (i, 0)),
            pl.BlockSpec((_BB,), lambda i, j: (i,)),
            pl.BlockSpec((_K, _CC, _D), lambda i, j: (0, j, 0)),
        ],
        out_specs=[
            pl.BlockSpec((_BB,), lambda i, j: (i,)),
            pl.BlockSpec((_BB,), lambda i, j: (i,)),
        ],
        out_shape=[
            jax.ShapeDtypeStruct((B,), jnp.float32),
            jax.ShapeDtypeStruct((B,), jnp.float32),
        ],
        compiler_params=pltpu.CompilerParams(
            dimension_semantics=("parallel", "arbitrary")),
    )(zn, y, pn)
    return (pos, neg)


# speedup vs baseline: 1.6649x; 1.6547x over previous
import jax
import jax.numpy as jnp
from jax.experimental import pallas as pl
from jax.experimental.pallas import tpu as pltpu

_C = 1024
_K = 8
_D = 256

_BB = 1024
_CC = 256


def _nrm(x):
    return x * jax.lax.rsqrt(
        jnp.maximum(jnp.sum(x * x, axis=1, keepdims=True), 1e-24))


def _normalize_kernel(z_ref, p_ref, zn_ref, pn_ref):
    zn_ref[...] = _nrm(z_ref[...]).astype(jnp.bfloat16)
    pt = _nrm(p_ref[...]).astype(jnp.bfloat16)
    pn_ref[...] = pt.reshape(_C, _K, _D).swapaxes(0, 1)


def _fused_kernel(zn_ref, y_ref, pn_ref, pos_ref, neg_ref):
    j = pl.program_id(1)
    zn = zn_ref[...]

    m = jax.lax.dot_general(
        pn_ref[0], zn, dimension_numbers=(((1,), (1,)), ((), ())),
        preferred_element_type=jnp.float32)
    for k in range(1, _K):
        m = jnp.maximum(m, jax.lax.dot_general(
            pn_ref[k], zn,
            dimension_numbers=(((1,), (1,)), ((), ())),
            preferred_element_type=jnp.float32))

    cls = j * _CC + jax.lax.broadcasted_iota(jnp.int32, (_CC, _BB), 0)
    same = cls == y_ref[...][None, :]

    ninf = jnp.float32(-jnp.inf)
    pos_c = jnp.max(jnp.where(same, m, ninf), axis=0)
    neg_c = jnp.max(jnp.where(same, ninf, m), axis=0)

    @pl.when(j == 0)
    def _init():
        pos_ref[...] = pos_c
        neg_ref[...] = neg_c

    @pl.when(j != 0)
    def _acc():
        pos_ref[...] = jnp.maximum(pos_ref[...], pos_c)
        neg_ref[...] = jnp.maximum(neg_ref[...], neg_c)


def kernel(z, y, protos):
    B = z.shape[0]
    zn, pn = pl.pallas_call(
        _normalize_kernel,
        out_shape=[
            jax.ShapeDtypeStruct((B, _D), jnp.bfloat16),
            jax.ShapeDtypeStruct((_K, _C, _D), jnp.bfloat16),
        ],
    )(z, protos.reshape(_C * _K, _D))

    grid = (B // _BB, _C // _CC)
    pos, neg = pl.pallas_call(
        _fused_kernel,
        grid=grid,
        in_specs=[
            pl.BlockSpec((_BB, _D), lambda i, j: (i, 0)),
            pl.BlockSpec((_BB,), lambda i, j: (i,)),
            pl.BlockSpec((_K, _CC, _D), lambda i, j: (0, j, 0)),
        ],
        out_specs=[
            pl.BlockSpec((_BB,), lambda i, j: (i,)),
            pl.BlockSpec((_BB,), lambda i, j: (i,)),
        ],
        out_shape=[
            jax.ShapeDtypeStruct((B,), jnp.float32),
            jax.ShapeDtypeStruct((B,), jnp.float32),
        ],
        compiler_params=pltpu.CompilerParams(
            dimension_semantics=("parallel", "arbitrary")),
    )(zn, y, pn)
    return (pos, neg)


# speedup vs baseline: 1.8086x; 1.0863x over previous
import jax
import jax.numpy as jnp
from jax.experimental import pallas as pl
from jax.experimental.pallas import tpu as pltpu

_C = 1024
_K = 8
_D = 256

_BB = 2048
_CC = 256


def _nrm(x):
    return x * jax.lax.rsqrt(
        jnp.maximum(jnp.sum(x * x, axis=1, keepdims=True), 1e-24))


def _normalize_kernel(z_ref, p_ref, zn_ref, pn_ref):
    zn_ref[...] = _nrm(z_ref[...]).astype(jnp.bfloat16)
    pt = _nrm(p_ref[...]).astype(jnp.bfloat16)
    pn_ref[...] = pt.reshape(_C, _K, _D).swapaxes(0, 1)


def _fused_kernel(zn_ref, y_ref, pn_ref, pos_ref, neg_ref):
    j = pl.program_id(1)
    zn = zn_ref[...]

    m = jax.lax.dot_general(
        pn_ref[0], zn, dimension_numbers=(((1,), (1,)), ((), ())),
        preferred_element_type=jnp.float32)
    for k in range(1, _K):
        m = jnp.maximum(m, jax.lax.dot_general(
            pn_ref[k], zn,
            dimension_numbers=(((1,), (1,)), ((), ())),
            preferred_element_type=jnp.float32))

    cls = j * _CC + jax.lax.broadcasted_iota(jnp.int32, (_CC, _BB), 0)
    same = cls == y_ref[...][None, :]

    ninf = jnp.float32(-jnp.inf)
    pos_c = jnp.max(jnp.where(same, m, ninf), axis=0)
    neg_c = jnp.max(jnp.where(same, ninf, m), axis=0)

    @pl.when(j == 0)
    def _init():
        pos_ref[...] = pos_c
        neg_ref[...] = neg_c

    @pl.when(j != 0)
    def _acc():
        pos_ref[...] = jnp.maximum(pos_ref[...], pos_c)
        neg_ref[...] = jnp.maximum(neg_ref[...], neg_c)


def kernel(z, y, protos):
    B = z.shape[0]
    zn, pn = pl.pallas_call(
        _normalize_kernel,
        out_shape=[
            jax.ShapeDtypeStruct((B, _D), jnp.bfloat16),
            jax.ShapeDtypeStruct((_K, _C, _D), jnp.bfloat16),
        ],
    )(z, protos.reshape(_C * _K, _D))

    grid = (B // _BB, _C // _CC)
    pos, neg = pl.pallas_call(
        _fused_kernel,
        grid=grid,
        in_specs=[
            pl.BlockSpec((_BB, _D), lambda i, j: (i, 0)),
            pl.BlockSpec((_BB,), lambda i, j: (i,)),
            pl.BlockSpec((_K, _CC, _D), lambda i, j: (0, j, 0)),
        ],
        out_specs=[
            pl.BlockSpec((_BB,), lambda i, j: (i,)),
            pl.BlockSpec((_BB,), lambda i, j: (i,)),
        ],
        out_shape=[
            jax.ShapeDtypeStruct((B,), jnp.float32),
            jax.ShapeDtypeStruct((B,), jnp.float32),
        ],
        compiler_params=pltpu.CompilerParams(
            dimension_semantics=("parallel", "arbitrary")),
    )(zn, y, pn)
    return (pos, neg)


# speedup vs baseline: 1.8696x; 1.0337x over previous
import jax
import jax.numpy as jnp
from jax.experimental import pallas as pl
from jax.experimental.pallas import tpu as pltpu

_C = 1024
_K = 8
_D = 256

_BB = 4096
_CC = 256


def _nrm(x):
    return x * jax.lax.rsqrt(
        jnp.maximum(jnp.sum(x * x, axis=1, keepdims=True), 1e-24))


def _normalize_kernel(z_ref, p_ref, zn_ref, pn_ref):
    zn_ref[...] = _nrm(z_ref[...]).astype(jnp.bfloat16)
    pt = _nrm(p_ref[...]).astype(jnp.bfloat16)
    pn_ref[...] = pt.reshape(_C, _K, _D).swapaxes(0, 1)


def _fused_kernel(zn_ref, y_ref, pn_ref, pos_ref, neg_ref):
    j = pl.program_id(1)
    zn = zn_ref[...]

    m = jax.lax.dot_general(
        pn_ref[0], zn, dimension_numbers=(((1,), (1,)), ((), ())),
        preferred_element_type=jnp.float32)
    for k in range(1, _K):
        m = jnp.maximum(m, jax.lax.dot_general(
            pn_ref[k], zn,
            dimension_numbers=(((1,), (1,)), ((), ())),
            preferred_element_type=jnp.float32))

    cls = j * _CC + jax.lax.broadcasted_iota(jnp.int32, (_CC, _BB), 0)
    same = cls == y_ref[...][None, :]

    ninf = jnp.float32(-jnp.inf)
    pos_c = jnp.max(jnp.where(same, m, ninf), axis=0)
    neg_c = jnp.max(jnp.where(same, ninf, m), axis=0)

    @pl.when(j == 0)
    def _init():
        pos_ref[...] = pos_c
        neg_ref[...] = neg_c

    @pl.when(j != 0)
    def _acc():
        pos_ref[...] = jnp.maximum(pos_ref[...], pos_c)
        neg_ref[...] = jnp.maximum(neg_ref[...], neg_c)


def kernel(z, y, protos):
    B = z.shape[0]
    zn, pn = pl.pallas_call(
        _normalize_kernel,
        out_shape=[
            jax.ShapeDtypeStruct((B, _D), jnp.bfloat16),
            jax.ShapeDtypeStruct((_K, _C, _D), jnp.bfloat16),
        ],
    )(z, protos.reshape(_C * _K, _D))

    grid = (B // _BB, _C // _CC)
    pos, neg = pl.pallas_call(
        _fused_kernel,
        grid=grid,
        in_specs=[
            pl.BlockSpec((_BB, _D), lambda i, j: (i, 0)),
            pl.BlockSpec((_BB,), lambda i, j: (i,)),
            pl.BlockSpec((_K, _CC, _D), lambda i, j: (0, j, 0)),
        ],
        out_specs=[
            pl.BlockSpec((_BB,), lambda i, j: (i,)),
            pl.BlockSpec((_BB,), lambda i, j: (i,)),
        ],
        out_shape=[
            jax.ShapeDtypeStruct((B,), jnp.float32),
            jax.ShapeDtypeStruct((B,), jnp.float32),
        ],
        compiler_params=pltpu.CompilerParams(
            dimension_semantics=("parallel", "arbitrary")),
    )(zn, y, pn)
    return (pos, neg)


# speedup vs baseline: 1.9283x; 1.0314x over previous
import jax
import jax.numpy as jnp
from jax.experimental import pallas as pl
from jax.experimental.pallas import tpu as pltpu

_C = 1024
_K = 8
_D = 256

_BB = 4096
_CC = 512


def _nrm(x):
    return x * jax.lax.rsqrt(
        jnp.maximum(jnp.sum(x * x, axis=1, keepdims=True), 1e-24))


def _normalize_kernel(z_ref, p_ref, zn_ref, pn_ref):
    zn_ref[...] = _nrm(z_ref[...]).astype(jnp.bfloat16)
    pt = _nrm(p_ref[...]).astype(jnp.bfloat16)
    pn_ref[...] = pt.reshape(_C, _K, _D).swapaxes(0, 1)


def _fused_kernel(zn_ref, y_ref, pn_ref, pos_ref, neg_ref):
    j = pl.program_id(1)
    zn = zn_ref[...]

    m = jax.lax.dot_general(
        pn_ref[0], zn, dimension_numbers=(((1,), (1,)), ((), ())),
        preferred_element_type=jnp.float32)
    for k in range(1, _K):
        m = jnp.maximum(m, jax.lax.dot_general(
            pn_ref[k], zn,
            dimension_numbers=(((1,), (1,)), ((), ())),
            preferred_element_type=jnp.float32))

    cls = j * _CC + jax.lax.broadcasted_iota(jnp.int32, (_CC, _BB), 0)
    same = cls == y_ref[...][None, :]

    ninf = jnp.float32(-jnp.inf)
    pos_c = jnp.max(jnp.where(same, m, ninf), axis=0)
    neg_c = jnp.max(jnp.where(same, ninf, m), axis=0)

    @pl.when(j == 0)
    def _init():
        pos_ref[...] = pos_c
        neg_ref[...] = neg_c

    @pl.when(j != 0)
    def _acc():
        pos_ref[...] = jnp.maximum(pos_ref[...], pos_c)
        neg_ref[...] = jnp.maximum(neg_ref[...], neg_c)


def kernel(z, y, protos):
    B = z.shape[0]
    zn, pn = pl.pallas_call(
        _normalize_kernel,
        out_shape=[
            jax.ShapeDtypeStruct((B, _D), jnp.bfloat16),
            jax.ShapeDtypeStruct((_K, _C, _D), jnp.bfloat16),
        ],
    )(z, protos.reshape(_C * _K, _D))

    grid = (B // _BB, _C // _CC)
    pos, neg = pl.pallas_call(
        _fused_kernel,
        grid=grid,
        in_specs=[
            pl.BlockSpec((_BB, _D), lambda i, j: (i, 0)),
            pl.BlockSpec((_BB,), lambda i, j: (i,)),
            pl.BlockSpec((_K, _CC, _D), lambda i, j: (0, j, 0)),
        ],
        out_specs=[
            pl.BlockSpec((_BB,), lambda i, j: (i,)),
            pl.BlockSpec((_BB,), lambda i, j: (i,)),
        ],
        out_shape=[
            jax.ShapeDtypeStruct((B,), jnp.float32),
            jax.ShapeDtypeStruct((B,), jnp.float32),
        ],
        compiler_params=pltpu.CompilerParams(
            dimension_semantics=("parallel", "arbitrary")),
    )(zn, y, pn)
    return (pos, neg)
